# Initial kernel scaffold; baseline (speedup 1.0000x reference)
import jax
import jax.numpy as jnp
from jax.experimental import pallas as pl


def kernel(x, edge_index, W1, b1, W2, b2):
    raise NotImplementedError("write your pallas kernel here")


# speedup vs baseline: 2.4604x; 2.4604x over previous
import jax
import jax.numpy as jnp
from jax import lax
from jax.experimental import pallas as pl
from jax.experimental.pallas import tpu as pltpu
from jax.experimental.pallas import tpu_sc as plsc

N_NODES = 10000
N_EDGES = 320000
D = 128

NC = 2
NS = 16
NW = NC * NS
CH = 128
NCHUNK = 80
EPW = NCHUNK * CH
E_PAD = NW * EPW
RPT = 632
N_ROWS = NS * RPT

_mesh = plsc.VectorSubcoreMesh(core_axis_name="c", subcore_axis_name="s")


def _tile_offs(sid):
    r0 = sid * RPT
    return [r0 + k * CH for k in range(RPT // CH)] + [r0 + RPT - CH]


def _seg_sum_body(table, src3, dst3, zrow, sums_out,
                  src_c, dst_c, rows_v, acc_sh, sem):
    cid = lax.axis_index("c")
    sid = lax.axis_index("s")
    wid = sid * NC + cid
    offs = _tile_offs(sid)

    pltpu.sync_copy(zrow, rows_v)
    for off in offs:
        pltpu.sync_copy(rows_v, acc_sh.at[pl.ds(off, CH)])

    plsc.subcore_barrier()

    def blk(c, carry):
        pltpu.sync_copy(src3.at[wid, c], src_c)
        pltpu.sync_copy(dst3.at[wid, c], dst_c)
        pltpu.async_copy(table.at[src_c], rows_v, sem).wait()
        pltpu.sync_copy(rows_v, acc_sh.at[dst_c], add=True)
        return carry

    lax.fori_loop(0, NCHUNK, blk, 0)

    plsc.subcore_barrier()

    for off in offs:
        pltpu.sync_copy(acc_sh.at[pl.ds(off, CH)], rows_v)
        pltpu.sync_copy(rows_v, sums_out.at[cid, pl.ds(off, CH)])


_seg_sum = pl.kernel(
    _seg_sum_body,
    out_type=jax.ShapeDtypeStruct((NC, N_ROWS, D), jnp.float32),
    mesh=_mesh,
    scratch_types=[
        pltpu.VMEM((CH,), jnp.int32),
        pltpu.VMEM((CH,), jnp.int32),
        pltpu.VMEM((CH, D), jnp.float32),
        pltpu.VMEM_SHARED((N_ROWS, D), jnp.float32),
        pltpu.SemaphoreType.DMA,
    ],
)


def _cnt_body(dst3, zrow, orow, cnt_out, dst_c, rows_v, acc_sh):
    cid = lax.axis_index("c")
    sid = lax.axis_index("s")
    wid = sid * NC + cid
    offs = _tile_offs(sid)

    pltpu.sync_copy(zrow, rows_v)
    for off in offs:
        pltpu.sync_copy(rows_v, acc_sh.at[pl.ds(off, CH)])

    plsc.subcore_barrier()

    pltpu.sync_copy(orow, rows_v)

    def blk(c, carry):
        pltpu.sync_copy(dst3.at[wid, c], dst_c)
        pltpu.sync_copy(rows_v, acc_sh.at[dst_c], add=True)
        return carry

    lax.fori_loop(0, NCHUNK, blk, 0)

    plsc.subcore_barrier()

    for off in offs:
        pltpu.sync_copy(acc_sh.at[pl.ds(off, CH)], rows_v)
        pltpu.sync_copy(rows_v, cnt_out.at[cid, pl.ds(off, CH)])


_cnt_sum = pl.kernel(
    _cnt_body,
    out_type=jax.ShapeDtypeStruct((NC, N_ROWS, D), jnp.float32),
    mesh=_mesh,
    scratch_types=[
        pltpu.VMEM((CH,), jnp.int32),
        pltpu.VMEM((CH, D), jnp.float32),
        pltpu.VMEM_SHARED((N_ROWS, D), jnp.float32),
    ],
)


_BLK = 128


def _mm_body(x_ref, w_ref, o_ref):
    o_ref[...] = lax.dot_general(
        x_ref[...], w_ref[...], (((1,), (1,)), ((), ())),
        preferred_element_type=jnp.float32)


_matmul = pl.pallas_call(
    _mm_body,
    out_shape=jax.ShapeDtypeStruct((N_ROWS, D), jnp.float32),
    grid=(N_ROWS // _BLK,),
    in_specs=[
        pl.BlockSpec((_BLK, D), lambda i: (i, 0)),
        pl.BlockSpec((D, D), lambda i: (0, 0)),
    ],
    out_specs=pl.BlockSpec((_BLK, D), lambda i: (i, 0)),
)


def _mid_body(s_ref, c_ref, b_ref, w_ref, o_ref):
    s = s_ref[0] + s_ref[1]
    c = (c_ref[0, :, 0] + c_ref[1, :, 0])[:, None]
    h = jnp.maximum(s / jnp.maximum(c, 1.0) + b_ref[...], 0.0)
    o_ref[...] = lax.dot_general(
        h, w_ref[...], (((1,), (1,)), ((), ())),
        preferred_element_type=jnp.float32)


_mid = pl.pallas_call(
    _mid_body,
    out_shape=jax.ShapeDtypeStruct((N_ROWS, D), jnp.float32),
    grid=(N_ROWS // _BLK,),
    in_specs=[
        pl.BlockSpec((NC, _BLK, D), lambda i: (0, i, 0)),
        pl.BlockSpec((NC, _BLK, D), lambda i: (0, i, 0)),
        pl.BlockSpec((1, D), lambda i: (0, 0)),
        pl.BlockSpec((D, D), lambda i: (0, 0)),
    ],
    out_specs=pl.BlockSpec((_BLK, D), lambda i: (i, 0)),
)


def _final_body(s_ref, c_ref, b_ref, o_ref):
    s = s_ref[0] + s_ref[1]
    c = (c_ref[0, :, 0] + c_ref[1, :, 0])[:, None]
    y = s / jnp.maximum(c, 1.0) + b_ref[...]
    m = jnp.max(y, axis=1, keepdims=True)
    lse = jnp.log(jnp.sum(jnp.exp(y - m), axis=1, keepdims=True)) + m
    o_ref[...] = y - lse


_final = pl.pallas_call(
    _final_body,
    out_shape=jax.ShapeDtypeStruct((N_ROWS, D), jnp.float32),
    grid=(N_ROWS // _BLK,),
    in_specs=[
        pl.BlockSpec((NC, _BLK, D), lambda i: (0, i, 0)),
        pl.BlockSpec((NC, _BLK, D), lambda i: (0, i, 0)),
        pl.BlockSpec((1, D), lambda i: (0, 0)),
    ],
    out_specs=pl.BlockSpec((_BLK, D), lambda i: (i, 0)),
)


def kernel(x, edge_index, W1, b1, W2, b2):
    ei = edge_index.astype(jnp.int32)
    pad = E_PAD - N_EDGES
    srcp = jnp.concatenate([ei[0], jnp.zeros((pad,), jnp.int32)])
    dstp = jnp.concatenate([ei[1], jnp.full((pad,), N_NODES, jnp.int32)])
    src3 = srcp.reshape(NW, NCHUNK, CH)
    dst3 = dstp.reshape(NW, NCHUNK, CH)
    zrow = jnp.zeros((CH, D), jnp.float32)
    orow = jnp.ones((CH, D), jnp.float32)
    xp = jnp.concatenate([x, jnp.zeros((N_ROWS - N_NODES, D), x.dtype)])

    cnt = _cnt_sum(dst3, zrow, orow)
    t1 = _matmul(xp, W1)
    s1 = _seg_sum(t1, src3, dst3, zrow)
    t2 = _mid(s1, cnt, b1.reshape(1, D), W2)
    s2 = _seg_sum(t2, src3, dst3, zrow)
    return _final(s2, cnt, b2.reshape(1, D))[:N_NODES]


# speedup vs baseline: 2.9167x; 1.1855x over previous
import jax
import jax.numpy as jnp
from jax import lax
from jax.experimental import pallas as pl
from jax.experimental.pallas import tpu as pltpu
from jax.experimental.pallas import tpu_sc as plsc

N_NODES = 10000
N_EDGES = 320000
D = 128

NC = 2
NS = 16
NW = NC * NS
CH = 128
NCHUNK = 80
EPW = NCHUNK * CH
E_PAD = NW * EPW
RPT = 632
N_ROWS = NS * RPT

_mesh = plsc.VectorSubcoreMesh(core_axis_name="c", subcore_axis_name="s")


def _tile_offs(sid):
    r0 = sid * RPT
    return [r0 + k * CH for k in range(RPT // CH)] + [r0 + RPT - CH]


def _seg_sum_body(table, src3, dst3, zrow, sums_out,
                  src_a, dst_a, rows_a, sem_a,
                  src_b, dst_b, rows_b, sem_b, acc_sh):
    cid = lax.axis_index("c")
    sid = lax.axis_index("s")
    wid = sid * NC + cid
    offs = _tile_offs(sid)

    pltpu.sync_copy(zrow, rows_a)
    for off in offs:
        pltpu.sync_copy(rows_a, acc_sh.at[pl.ds(off, CH)])

    plsc.subcore_barrier()

    bufs = [(src_a, dst_a, rows_a, sem_a), (src_b, dst_b, rows_b, sem_b)]
    handles = {}
    pltpu.sync_copy(src3.at[wid, 0], src_a)
    handles[0] = pltpu.async_copy(table.at[src_a], rows_a, sem_a)
    for c in range(NCHUNK):
        srcb, dstb, rows, _ = bufs[c % 2]
        if c + 1 < NCHUNK:
            nsrc, _, nrows, nsem = bufs[(c + 1) % 2]
            pltpu.sync_copy(src3.at[wid, c + 1], nsrc)
            handles[c + 1] = pltpu.async_copy(table.at[nsrc], nrows, nsem)
        pltpu.sync_copy(dst3.at[wid, c], dstb)
        handles[c].wait()
        pltpu.sync_copy(rows, acc_sh.at[dstb], add=True)

    plsc.subcore_barrier()

    for off in offs:
        pltpu.sync_copy(acc_sh.at[pl.ds(off, CH)], rows_a)
        pltpu.sync_copy(rows_a, sums_out.at[cid, pl.ds(off, CH)])


_seg_sum = pl.kernel(
    _seg_sum_body,
    out_type=jax.ShapeDtypeStruct((NC, N_ROWS, D), jnp.float32),
    mesh=_mesh,
    scratch_types=[
        pltpu.VMEM((CH,), jnp.int32),
        pltpu.VMEM((CH,), jnp.int32),
        pltpu.VMEM((CH, D), jnp.float32),
        pltpu.SemaphoreType.DMA,
        pltpu.VMEM((CH,), jnp.int32),
        pltpu.VMEM((CH,), jnp.int32),
        pltpu.VMEM((CH, D), jnp.float32),
        pltpu.SemaphoreType.DMA,
        pltpu.VMEM_SHARED((N_ROWS, D), jnp.float32),
    ],
)


def _cnt_body(dst3, zrow, orow, cnt_out, dst_c, rows_v, acc_sh):
    cid = lax.axis_index("c")
    sid = lax.axis_index("s")
    wid = sid * NC + cid
    offs = _tile_offs(sid)

    pltpu.sync_copy(zrow, rows_v)
    for off in offs:
        pltpu.sync_copy(rows_v, acc_sh.at[pl.ds(off, CH)])

    plsc.subcore_barrier()

    pltpu.sync_copy(orow, rows_v)

    def blk(c, carry):
        pltpu.sync_copy(dst3.at[wid, c], dst_c)
        pltpu.sync_copy(rows_v, acc_sh.at[dst_c], add=True)
        return carry

    lax.fori_loop(0, NCHUNK, blk, 0)

    plsc.subcore_barrier()

    for off in offs:
        pltpu.sync_copy(acc_sh.at[pl.ds(off, CH)], rows_v)
        pltpu.sync_copy(rows_v, cnt_out.at[cid, pl.ds(off, CH)])


_cnt_sum = pl.kernel(
    _cnt_body,
    out_type=jax.ShapeDtypeStruct((NC, N_ROWS, D), jnp.float32),
    mesh=_mesh,
    scratch_types=[
        pltpu.VMEM((CH,), jnp.int32),
        pltpu.VMEM((CH, D), jnp.float32),
        pltpu.VMEM_SHARED((N_ROWS, D), jnp.float32),
    ],
)


_BLK = 128


def _mm_body(x_ref, w_ref, o_ref):
    o_ref[...] = lax.dot_general(
        x_ref[...], w_ref[...], (((1,), (1,)), ((), ())),
        preferred_element_type=jnp.float32)


_matmul = pl.pallas_call(
    _mm_body,
    out_shape=jax.ShapeDtypeStruct((N_ROWS, D), jnp.float32),
    grid=(N_ROWS // _BLK,),
    in_specs=[
        pl.BlockSpec((_BLK, D), lambda i: (i, 0)),
        pl.BlockSpec((D, D), lambda i: (0, 0)),
    ],
    out_specs=pl.BlockSpec((_BLK, D), lambda i: (i, 0)),
)


def _mid_body(s_ref, c_ref, b_ref, w_ref, o_ref):
    s = s_ref[0] + s_ref[1]
    c = (c_ref[0, :, 0] + c_ref[1, :, 0])[:, None]
    h = jnp.maximum(s / jnp.maximum(c, 1.0) + b_ref[...], 0.0)
    o_ref[...] = lax.dot_general(
        h, w_ref[...], (((1,), (1,)), ((), ())),
        preferred_element_type=jnp.float32)


_mid = pl.pallas_call(
    _mid_body,
    out_shape=jax.ShapeDtypeStruct((N_ROWS, D), jnp.float32),
    grid=(N_ROWS // _BLK,),
    in_specs=[
        pl.BlockSpec((NC, _BLK, D), lambda i: (0, i, 0)),
        pl.BlockSpec((NC, _BLK, D), lambda i: (0, i, 0)),
        pl.BlockSpec((1, D), lambda i: (0, 0)),
        pl.BlockSpec((D, D), lambda i: (0, 0)),
    ],
    out_specs=pl.BlockSpec((_BLK, D), lambda i: (i, 0)),
)


def _final_body(s_ref, c_ref, b_ref, o_ref):
    s = s_ref[0] + s_ref[1]
    c = (c_ref[0, :, 0] + c_ref[1, :, 0])[:, None]
    y = s / jnp.maximum(c, 1.0) + b_ref[...]
    m = jnp.max(y, axis=1, keepdims=True)
    lse = jnp.log(jnp.sum(jnp.exp(y - m), axis=1, keepdims=True)) + m
    o_ref[...] = y - lse


_final = pl.pallas_call(
    _final_body,
    out_shape=jax.ShapeDtypeStruct((N_ROWS, D), jnp.float32),
    grid=(N_ROWS // _BLK,),
    in_specs=[
        pl.BlockSpec((NC, _BLK, D), lambda i: (0, i, 0)),
        pl.BlockSpec((NC, _BLK, D), lambda i: (0, i, 0)),
        pl.BlockSpec((1, D), lambda i: (0, 0)),
    ],
    out_specs=pl.BlockSpec((_BLK, D), lambda i: (i, 0)),
)


def kernel(x, edge_index, W1, b1, W2, b2):
    ei = edge_index.astype(jnp.int32)
    pad = E_PAD - N_EDGES
    srcp = jnp.concatenate([ei[0], jnp.zeros((pad,), jnp.int32)])
    dstp = jnp.concatenate([ei[1], jnp.full((pad,), N_NODES, jnp.int32)])
    src3 = srcp.reshape(NW, NCHUNK, CH)
    dst3 = dstp.reshape(NW, NCHUNK, CH)
    zrow = jnp.zeros((CH, D), jnp.float32)
    orow = jnp.ones((CH, D), jnp.float32)
    xp = jnp.concatenate([x, jnp.zeros((N_ROWS - N_NODES, D), x.dtype)])

    cnt = _cnt_sum(dst3, zrow, orow)
    t1 = _matmul(xp, W1)
    s1 = _seg_sum(t1, src3, dst3, zrow)
    t2 = _mid(s1, cnt, b1.reshape(1, D), W2)
    s2 = _seg_sum(t2, src3, dst3, zrow)
    return _final(s2, cnt, b2.reshape(1, D))[:N_NODES]


# speedup vs baseline: 2.9892x; 1.0249x over previous
import jax
import jax.numpy as jnp
from jax import lax
from jax.experimental import pallas as pl
from jax.experimental.pallas import tpu as pltpu
from jax.experimental.pallas import tpu_sc as plsc

N_NODES = 10000
N_EDGES = 320000
D = 128

NC = 2
NS = 16
NW = NC * NS
CH = 128
NCHUNK = 80
EPW = NCHUNK * CH
E_PAD = NW * EPW
RPT = 632
N_ROWS = NS * RPT

_mesh = plsc.VectorSubcoreMesh(core_axis_name="c", subcore_axis_name="s")


def _tile_offs(sid):
    r0 = sid * RPT
    return [r0 + k * CH for k in range(RPT // CH)] + [r0 + RPT - CH]


def _seg_sum_body(table, src3, dst3, zrow, sums_out,
                  src_a, dst_a, rows_a, sem_a, ssem_a,
                  src_b, dst_b, rows_b, sem_b, ssem_b, acc_sh):
    cid = lax.axis_index("c")
    sid = lax.axis_index("s")
    wid = sid * NC + cid
    offs = _tile_offs(sid)

    pltpu.sync_copy(zrow, rows_a)
    for off in offs:
        pltpu.sync_copy(rows_a, acc_sh.at[pl.ds(off, CH)])

    plsc.subcore_barrier()

    bufs = [(src_a, dst_a, rows_a, sem_a, ssem_a),
            (src_b, dst_b, rows_b, sem_b, ssem_b)]
    gath = {}
    scat = {}
    pltpu.sync_copy(src3.at[wid, 0], src_a)
    gath[0] = pltpu.async_copy(table.at[src_a], rows_a, sem_a)
    for c in range(NCHUNK):
        srcb, dstb, rows, _, ssem = bufs[c % 2]
        if c + 1 < NCHUNK:
            nsrc, _, nrows, nsem, _ = bufs[(c + 1) % 2]
            if c - 1 >= 0:
                scat[c - 1].wait()
            pltpu.sync_copy(src3.at[wid, c + 1], nsrc)
            gath[c + 1] = pltpu.async_copy(table.at[nsrc], nrows, nsem)
        pltpu.sync_copy(dst3.at[wid, c], dstb)
        gath[c].wait()
        scat[c] = pltpu.async_copy(rows, acc_sh.at[dstb], ssem, add=True)
    scat[NCHUNK - 2].wait()
    scat[NCHUNK - 1].wait()

    plsc.subcore_barrier()

    for off in offs:
        pltpu.sync_copy(acc_sh.at[pl.ds(off, CH)], rows_a)
        pltpu.sync_copy(rows_a, sums_out.at[cid, pl.ds(off, CH)])


_seg_sum = pl.kernel(
    _seg_sum_body,
    out_type=jax.ShapeDtypeStruct((NC, N_ROWS, D), jnp.float32),
    mesh=_mesh,
    scratch_types=[
        pltpu.VMEM((CH,), jnp.int32),
        pltpu.VMEM((CH,), jnp.int32),
        pltpu.VMEM((CH, D), jnp.float32),
        pltpu.SemaphoreType.DMA,
        pltpu.SemaphoreType.DMA,
        pltpu.VMEM((CH,), jnp.int32),
        pltpu.VMEM((CH,), jnp.int32),
        pltpu.VMEM((CH, D), jnp.float32),
        pltpu.SemaphoreType.DMA,
        pltpu.SemaphoreType.DMA,
        pltpu.VMEM_SHARED((N_ROWS, D), jnp.float32),
    ],
)


def _cnt_body(dst3, zrow, orow, cnt_out,
              dst_a, ssem_a, dst_b, ssem_b, rows_v, acc_sh):
    cid = lax.axis_index("c")
    sid = lax.axis_index("s")
    wid = sid * NC + cid
    offs = _tile_offs(sid)

    pltpu.sync_copy(zrow, rows_v)
    for off in offs:
        pltpu.sync_copy(rows_v, acc_sh.at[pl.ds(off, CH)])

    plsc.subcore_barrier()

    pltpu.sync_copy(orow, rows_v)

    bufs = [(dst_a, ssem_a), (dst_b, ssem_b)]
    scat = {}
    for c in range(NCHUNK):
        dstb, ssem = bufs[c % 2]
        if c - 2 >= 0:
            scat[c - 2].wait()
        pltpu.sync_copy(dst3.at[wid, c], dstb)
        scat[c] = pltpu.async_copy(rows_v, acc_sh.at[dstb], ssem, add=True)
    scat[NCHUNK - 2].wait()
    scat[NCHUNK - 1].wait()

    plsc.subcore_barrier()

    for off in offs:
        pltpu.sync_copy(acc_sh.at[pl.ds(off, CH)], rows_v)
        pltpu.sync_copy(rows_v, cnt_out.at[cid, pl.ds(off, CH)])


_cnt_sum = pl.kernel(
    _cnt_body,
    out_type=jax.ShapeDtypeStruct((NC, N_ROWS, D), jnp.float32),
    mesh=_mesh,
    scratch_types=[
        pltpu.VMEM((CH,), jnp.int32),
        pltpu.SemaphoreType.DMA,
        pltpu.VMEM((CH,), jnp.int32),
        pltpu.SemaphoreType.DMA,
        pltpu.VMEM((CH, D), jnp.float32),
        pltpu.VMEM_SHARED((N_ROWS, D), jnp.float32),
    ],
)


_BLK = 128


def _mm_body(x_ref, w_ref, o_ref):
    o_ref[...] = lax.dot_general(
        x_ref[...], w_ref[...], (((1,), (1,)), ((), ())),
        preferred_element_type=jnp.float32)


_matmul = pl.pallas_call(
    _mm_body,
    out_shape=jax.ShapeDtypeStruct((N_ROWS, D), jnp.float32),
    grid=(N_ROWS // _BLK,),
    in_specs=[
        pl.BlockSpec((_BLK, D), lambda i: (i, 0)),
        pl.BlockSpec((D, D), lambda i: (0, 0)),
    ],
    out_specs=pl.BlockSpec((_BLK, D), lambda i: (i, 0)),
)


def _mid_body(s_ref, c_ref, b_ref, w_ref, o_ref):
    s = s_ref[0] + s_ref[1]
    c = (c_ref[0, :, 0] + c_ref[1, :, 0])[:, None]
    h = jnp.maximum(s / jnp.maximum(c, 1.0) + b_ref[...], 0.0)
    o_ref[...] = lax.dot_general(
        h, w_ref[...], (((1,), (1,)), ((), ())),
        preferred_element_type=jnp.float32)


_mid = pl.pallas_call(
    _mid_body,
    out_shape=jax.ShapeDtypeStruct((N_ROWS, D), jnp.float32),
    grid=(N_ROWS // _BLK,),
    in_specs=[
        pl.BlockSpec((NC, _BLK, D), lambda i: (0, i, 0)),
        pl.BlockSpec((NC, _BLK, D), lambda i: (0, i, 0)),
        pl.BlockSpec((1, D), lambda i: (0, 0)),
        pl.BlockSpec((D, D), lambda i: (0, 0)),
    ],
    out_specs=pl.BlockSpec((_BLK, D), lambda i: (i, 0)),
)


def _final_body(s_ref, c_ref, b_ref, o_ref):
    s = s_ref[0] + s_ref[1]
    c = (c_ref[0, :, 0] + c_ref[1, :, 0])[:, None]
    y = s / jnp.maximum(c, 1.0) + b_ref[...]
    m = jnp.max(y, axis=1, keepdims=True)
    lse = jnp.log(jnp.sum(jnp.exp(y - m), axis=1, keepdims=True)) + m
    o_ref[...] = y - lse


_final = pl.pallas_call(
    _final_body,
    out_shape=jax.ShapeDtypeStruct((N_ROWS, D), jnp.float32),
    grid=(N_ROWS // _BLK,),
    in_specs=[
        pl.BlockSpec((NC, _BLK, D), lambda i: (0, i, 0)),
        pl.BlockSpec((NC, _BLK, D), lambda i: (0, i, 0)),
        pl.BlockSpec((1, D), lambda i: (0, 0)),
    ],
    out_specs=pl.BlockSpec((_BLK, D), lambda i: (i, 0)),
)


def kernel(x, edge_index, W1, b1, W2, b2):
    ei = edge_index.astype(jnp.int32)
    pad = E_PAD - N_EDGES
    srcp = jnp.concatenate([ei[0], jnp.zeros((pad,), jnp.int32)])
    dstp = jnp.concatenate([ei[1], jnp.full((pad,), N_NODES, jnp.int32)])
    src3 = srcp.reshape(NW, NCHUNK, CH)
    dst3 = dstp.reshape(NW, NCHUNK, CH)
    zrow = jnp.zeros((CH, D), jnp.float32)
    orow = jnp.ones((CH, D), jnp.float32)
    xp = jnp.concatenate([x, jnp.zeros((N_ROWS - N_NODES, D), x.dtype)])

    cnt = _cnt_sum(dst3, zrow, orow)
    t1 = _matmul(xp, W1)
    s1 = _seg_sum(t1, src3, dst3, zrow)
    t2 = _mid(s1, cnt, b1.reshape(1, D), W2)
    s2 = _seg_sum(t2, src3, dst3, zrow)
    return _final(s2, cnt, b2.reshape(1, D))[:N_NODES]
